# baseline (speedup 1.0000x reference)
import functools

import jax
import jax.numpy as jnp
from jax import lax
from jax.experimental import pallas as pl
from jax.experimental.pallas import tpu as pltpu
from jax.experimental.pallas import tpu_sc as plsc

B = 64
NUM_NODES = 196
N = B * NUM_NODES
E = 401408
LATENT = 128
NEG = 0.2

NUM_TILES = 32
EPT = E // NUM_TILES
BLK = 3136
NBLK = E // BLK
FPT = LATENT // NUM_TILES
ROWBLK = 896


def _leaky(x):
    return jnp.where(x > 0, x, NEG * x)


def _fc_body(z_ref, w_ref, b_ref, o_ref):
    acc = lax.dot_general(z_ref[...], w_ref[...],
                          (((1,), (1,)), ((), ())),
                          preferred_element_type=jnp.float32)
    o_ref[...] = jnp.maximum(acc + b_ref[...], 0.0)


def _fc(z, fc_w, fc_b):
    nblk = 49
    blk = (NUM_NODES * LATENT) // nblk
    return pl.pallas_call(
        _fc_body,
        grid=(nblk,),
        in_specs=[
            pl.BlockSpec((B, LATENT), lambda i: (0, 0)),
            pl.BlockSpec((blk, LATENT), lambda i: (i, 0)),
            pl.BlockSpec((1, blk), lambda i: (0, i)),
        ],
        out_specs=pl.BlockSpec((B, blk), lambda i: (0, i)),
        out_shape=jax.ShapeDtypeStruct((B, NUM_NODES * LATENT), jnp.float32),
    )(z, fc_w, fc_b.reshape(1, -1))


def _pre_body(transposed, x_ref, w_ref, asrc_ref, adst_ref,
              ht_ref, s_ref, d_ref, sm_ref):
    i = pl.program_id(0)
    if transposed:
        ht = lax.dot_general(w_ref[...], x_ref[...],
                             (((0,), (0,)), ((), ())),
                             preferred_element_type=jnp.float32)
    else:
        ht = lax.dot_general(w_ref[...], x_ref[...],
                             (((0,), (1,)), ((), ())),
                             preferred_element_type=jnp.float32)
    ht_ref[...] = ht
    s = jnp.sum(ht * asrc_ref[...], axis=0, keepdims=True)
    d = jnp.sum(ht * adst_ref[...], axis=0, keepdims=True)
    s_ref[...] = s
    d_ref[...] = d

    @pl.when(i == 0)
    def _():
        sm_ref[...] = jnp.full((1, 1), -jnp.inf, jnp.float32)

    sm_ref[...] = jnp.maximum(sm_ref[...], jnp.max(s))


def _pre(x, w, a_src, a_dst, transposed):
    nblk = N // ROWBLK
    if transposed:
        x_spec = pl.BlockSpec((LATENT, ROWBLK), lambda i: (0, i))
    else:
        x_spec = pl.BlockSpec((ROWBLK, LATENT), lambda i: (i, 0))
    return pl.pallas_call(
        functools.partial(_pre_body, transposed),
        grid=(nblk,),
        in_specs=[
            x_spec,
            pl.BlockSpec((LATENT, LATENT), lambda i: (0, 0)),
            pl.BlockSpec((LATENT, 1), lambda i: (0, 0)),
            pl.BlockSpec((LATENT, 1), lambda i: (0, 0)),
        ],
        out_specs=[
            pl.BlockSpec((LATENT, ROWBLK), lambda i: (0, i)),
            pl.BlockSpec((1, ROWBLK), lambda i: (0, i)),
            pl.BlockSpec((1, ROWBLK), lambda i: (0, i)),
            pl.BlockSpec((1, 1), lambda i: (0, 0)),
        ],
        out_shape=[
            jax.ShapeDtypeStruct((LATENT, N), jnp.float32),
            jax.ShapeDtypeStruct((1, N), jnp.float32),
            jax.ShapeDtypeStruct((1, N), jnp.float32),
            jax.ShapeDtypeStruct((1, 1), jnp.float32),
        ],
    )(x, w, a_src.reshape(-1, 1), a_dst.reshape(-1, 1))


def _combine_body(relu, transpose_out, pt_ref, den_ref, ht_ref, s_ref, d_ref,
                  sm_ref, b_ref, o_ref):
    s = s_ref[...]
    d = d_ref[...]
    smax = sm_ref[0, 0]
    m = _leaky(smax + d)
    ea_self = jnp.exp(_leaky(s + d) - m)
    den_tot = jnp.sum(den_ref[...], axis=0, keepdims=True) + ea_self
    num = pt_ref[...] + ea_self * ht_ref[...]
    out = num / den_tot + b_ref[...]
    if relu:
        out = jnp.maximum(out, 0.0)
    if transpose_out:
        o_ref[...] = out.T
    else:
        o_ref[...] = out


def _combine(pt, den, ht, s, d, smax, bias, relu, transpose_out):
    nblk = N // ROWBLK
    if transpose_out:
        out_spec = pl.BlockSpec((ROWBLK, LATENT), lambda i: (i, 0))
        out_shape = jax.ShapeDtypeStruct((N, LATENT), jnp.float32)
    else:
        out_spec = pl.BlockSpec((LATENT, ROWBLK), lambda i: (0, i))
        out_shape = jax.ShapeDtypeStruct((LATENT, N), jnp.float32)
    return pl.pallas_call(
        functools.partial(_combine_body, relu, transpose_out),
        grid=(nblk,),
        in_specs=[
            pl.BlockSpec((LATENT, ROWBLK), lambda i: (0, i)),
            pl.BlockSpec((NUM_TILES, ROWBLK), lambda i: (0, i)),
            pl.BlockSpec((LATENT, ROWBLK), lambda i: (0, i)),
            pl.BlockSpec((1, ROWBLK), lambda i: (0, i)),
            pl.BlockSpec((1, ROWBLK), lambda i: (0, i)),
            pl.BlockSpec((1, 1), lambda i: (0, 0)),
            pl.BlockSpec((LATENT, 1), lambda i: (0, 0)),
        ],
        out_specs=out_spec,
        out_shape=out_shape,
    )(pt, den, ht, s, d, smax, bias.reshape(-1, 1))


def _ew_body(s_hbm, d_hbm, sm_hbm, src_hbm, dst_hbm,
             rec_hbm, den_hbm,
             s_v, d_v, den_v, sm_v, stg):
    c = lax.axis_index("c")
    sub = lax.axis_index("s")
    wid = c * 16 + sub
    base = wid * EPT

    pltpu.sync_copy(s_hbm, s_v)
    pltpu.sync_copy(d_hbm, d_v)
    pltpu.sync_copy(sm_hbm, sm_v)
    for j in range(EPT // BLK):
        pltpu.sync_copy(src_hbm.at[pl.ds(base + j * BLK, BLK)],
                        stg.at[pl.ds(j * 3 * BLK, BLK)])
        pltpu.sync_copy(dst_hbm.at[pl.ds(base + j * BLK, BLK)],
                        stg.at[pl.ds(j * 3 * BLK + BLK, BLK)])

    zero16 = jnp.zeros((16,), jnp.float32)

    def _zden(i, carry):
        den_v[pl.ds(i * 16, 16)] = zero16
        return carry
    lax.fori_loop(0, N // 16, _zden, 0)

    smax = sm_v[...]

    UNROLL = 4
    for j in range(EPT // BLK):
        def _group(i, carry):
            for u in range(UNROLL):
                o = (i * UNROLL + u) * 16
                isrc = stg[pl.ds(j * 3 * BLK + o, 16)]
                idst = stg[pl.ds(j * 3 * BLK + BLK + o, 16)]
                sv = plsc.load_gather(s_v, [isrc])
                dv = plsc.load_gather(d_v, [idst])
                t = smax + dv
                m = jnp.where(t > 0, t, NEG * t)
                a = sv + dv
                a = jnp.where(a > 0, a, NEG * a)
                ea = jnp.exp(a - m)
                plsc.addupdate_scatter(den_v, [idst], ea)
                stg[pl.ds(j * 3 * BLK + 2 * BLK + o, 16)] = (
                    plsc.bitcast(ea, jnp.int32))
            return carry
        lax.fori_loop(0, BLK // (16 * UNROLL), _group, 0)

    pltpu.sync_copy(stg, rec_hbm.at[pl.ds(wid * 3 * EPT, 3 * EPT)])
    pltpu.sync_copy(den_v, den_hbm.at[wid])


def _edge_weights(s, d, smax16, src, dst):
    mesh = plsc.VectorSubcoreMesh(core_axis_name="c", subcore_axis_name="s")
    f = pl.kernel(
        _ew_body,
        out_type=(
            jax.ShapeDtypeStruct((3 * E,), jnp.int32),
            jax.ShapeDtypeStruct((NUM_TILES, N), jnp.float32),
        ),
        mesh=mesh,
        scratch_types=(
            pltpu.VMEM((N,), jnp.float32),
            pltpu.VMEM((N,), jnp.float32),
            pltpu.VMEM((N,), jnp.float32),
            pltpu.VMEM((16,), jnp.float32),
            pltpu.VMEM((3 * EPT,), jnp.int32),
        ),
        compiler_params=pltpu.CompilerParams(needs_layout_passes=False),
    )
    return f(s, d, smax16, src, dst)


def _rows_body(ht_hbm, rec_hbm, out_hbm,
               h_part, acc, rec_a, rec_b, isem_a, isem_b):
    c = lax.axis_index("c")
    sub = lax.axis_index("s")
    wid = c * 16 + sub
    frow = wid * FPT

    pltpu.sync_copy(ht_hbm.at[pl.ds(frow, FPT)], h_part)

    zero16 = jnp.zeros((16,), jnp.float32)

    def _zacc(i, carry):
        for ff in range(FPT):
            acc[ff, pl.ds(i * 16, 16)] = zero16
        return carry
    lax.fori_loop(0, N // 16, _zacc, 0)

    pltpu.async_copy(rec_hbm.at[pl.ds(0, 3 * BLK)], rec_a, isem_a)
    pltpu.async_copy(rec_hbm.at[pl.ds(3 * BLK, 3 * BLK)], rec_b, isem_b)

    f16 = [lax.broadcast(ff, (16,)) for ff in range(FPT)]

    def _do_block(rec_v):
        UNROLL = 2
        def _group(i, carry):
            for u in range(UNROLL):
                o = (i * UNROLL + u) * 16
                isrc = rec_v[pl.ds(o, 16)]
                idst = rec_v[pl.ds(BLK + o, 16)]
                ea = plsc.bitcast(rec_v[pl.ds(2 * BLK + o, 16)], jnp.float32)
                for ff in range(FPT):
                    val = plsc.load_gather(h_part, [f16[ff], isrc]) * ea
                    plsc.addupdate_scatter(acc, [f16[ff], idst], val)
            return carry
        lax.fori_loop(0, BLK // (16 * UNROLL), _group, 0)

    def _pair(jj, carry):
        for u, (rec_v, isem) in enumerate(((rec_a, isem_a), (rec_b, isem_b))):
            j = jj * 2 + u
            pltpu.make_async_copy(rec_hbm.at[pl.ds(0, 3 * BLK)], rec_v,
                                  isem).wait()
            _do_block(rec_v)

            @pl.when(j + 2 < NBLK)
            def _():
                pltpu.async_copy(rec_hbm.at[pl.ds((j + 2) * 3 * BLK, 3 * BLK)],
                                 rec_v, isem)
        return carry

    lax.fori_loop(0, NBLK // 2, _pair, 0)

    pltpu.sync_copy(acc, out_hbm.at[pl.ds(frow, FPT)])


def _edge_rows(ht, rec):
    mesh = plsc.VectorSubcoreMesh(core_axis_name="c", subcore_axis_name="s")
    f = pl.kernel(
        _rows_body,
        out_type=jax.ShapeDtypeStruct((LATENT, N), jnp.float32),
        mesh=mesh,
        scratch_types=(
            pltpu.VMEM((FPT, N), jnp.float32),
            pltpu.VMEM((FPT, N), jnp.float32),
            pltpu.VMEM((3 * BLK,), jnp.int32),
            pltpu.VMEM((3 * BLK,), jnp.int32),
            pltpu.SemaphoreType.DMA,
            pltpu.SemaphoreType.DMA,
        ),
        compiler_params=pltpu.CompilerParams(needs_layout_passes=False),
    )
    return f(ht, rec)


def kernel(z, edge_index, params):
    src = edge_index[0]
    dst = edge_index[1]
    x = _fc(z, params["fc_W"], params["fc_b"]).reshape(N, LATENT)
    convs = params["convs"]
    transposed = False
    for i, p in enumerate(convs):
        last = i == len(convs) - 1
        ht, s, d, smax = _pre(x, p["W"], p["a_src"], p["a_dst"], transposed)
        smax16 = jnp.broadcast_to(smax.reshape(1), (16,))
        rec, den = _edge_weights(s.reshape(N), d.reshape(N), smax16, src, dst)
        pt = _edge_rows(ht, rec)
        x = _combine(pt, den, ht, s, d, smax, p["b"],
                     relu=not last, transpose_out=last)
        transposed = True
    return x


# speedup vs baseline: 2.8166x; 2.8166x over previous
import functools

import jax
import jax.numpy as jnp
from jax import lax
from jax.experimental import pallas as pl
from jax.experimental.pallas import tpu as pltpu
from jax.experimental.pallas import tpu_sc as plsc

B = 64
NUM_NODES = 196
N = B * NUM_NODES
E = 401408
LATENT = 128
HALF = LATENT // 2
NEG = 0.2

NUM_TILES = 32
EPT = E // NUM_TILES
CHUNK = 64
REC = 3 * CHUNK
NC = EPT // CHUNK
ROWBLK = 896


def _leaky(x):
    return jnp.where(x > 0, x, NEG * x)


def _fc_body(z_ref, w_ref, b_ref, o_ref):
    acc = lax.dot_general(z_ref[...], w_ref[...],
                          (((1,), (1,)), ((), ())),
                          preferred_element_type=jnp.float32)
    o_ref[...] = jnp.maximum(acc + b_ref[...], 0.0)


def _fc(z, fc_w, fc_b):
    nblk = 49
    blk = (NUM_NODES * LATENT) // nblk
    return pl.pallas_call(
        _fc_body,
        grid=(nblk,),
        in_specs=[
            pl.BlockSpec((B, LATENT), lambda i: (0, 0)),
            pl.BlockSpec((blk, LATENT), lambda i: (i, 0)),
            pl.BlockSpec((1, blk), lambda i: (0, i)),
        ],
        out_specs=pl.BlockSpec((B, blk), lambda i: (0, i)),
        out_shape=jax.ShapeDtypeStruct((B, NUM_NODES * LATENT), jnp.float32),
    )(z, fc_w, fc_b.reshape(1, -1))


def _pre_body(x_ref, w_ref, asrc_ref, adst_ref,
              h_ref, s_ref, d_ref, sm_ref):
    i = pl.program_id(0)
    h = lax.dot_general(x_ref[...], w_ref[...],
                        (((1,), (0,)), ((), ())),
                        preferred_element_type=jnp.float32)
    h_ref[...] = h
    s = jnp.sum(h * asrc_ref[...], axis=1, keepdims=True)
    d = jnp.sum(h * adst_ref[...], axis=1, keepdims=True)
    s_ref[...] = s
    d_ref[...] = d

    @pl.when(i == 0)
    def _():
        sm_ref[...] = jnp.full((1, 1), -jnp.inf, jnp.float32)

    sm_ref[...] = jnp.maximum(sm_ref[...], jnp.max(s))


def _pre(x, w, a_src, a_dst):
    nblk = N // ROWBLK
    return pl.pallas_call(
        _pre_body,
        grid=(nblk,),
        in_specs=[
            pl.BlockSpec((ROWBLK, LATENT), lambda i: (i, 0)),
            pl.BlockSpec((LATENT, LATENT), lambda i: (0, 0)),
            pl.BlockSpec((1, LATENT), lambda i: (0, 0)),
            pl.BlockSpec((1, LATENT), lambda i: (0, 0)),
        ],
        out_specs=[
            pl.BlockSpec((ROWBLK, LATENT), lambda i: (i, 0)),
            pl.BlockSpec((ROWBLK, 1), lambda i: (i, 0)),
            pl.BlockSpec((ROWBLK, 1), lambda i: (i, 0)),
            pl.BlockSpec((1, 1), lambda i: (0, 0)),
        ],
        out_shape=[
            jax.ShapeDtypeStruct((N, LATENT), jnp.float32),
            jax.ShapeDtypeStruct((N, 1), jnp.float32),
            jax.ShapeDtypeStruct((N, 1), jnp.float32),
            jax.ShapeDtypeStruct((1, 1), jnp.float32),
        ],
    )(x, w, a_src.reshape(1, -1), a_dst.reshape(1, -1))


def _combine_body(relu, p_ref, den_ref, h_ref, s_ref, d_ref,
                  sm_ref, b_ref, o_ref):
    s = s_ref[...]
    d = d_ref[...]
    smax = sm_ref[0, 0]
    m = _leaky(smax + d)
    ea_self = jnp.exp(_leaky(s + d) - m)
    den_tot = jnp.sum(den_ref[...], axis=0)[:, None] + ea_self
    num = p_ref[0] + p_ref[1] + ea_self * h_ref[...]
    out = num / den_tot + b_ref[...]
    if relu:
        out = jnp.maximum(out, 0.0)
    o_ref[...] = out


def _combine(p, den, h, s, d, smax, bias, relu):
    nblk = N // ROWBLK
    return pl.pallas_call(
        functools.partial(_combine_body, relu),
        grid=(nblk,),
        in_specs=[
            pl.BlockSpec((2, ROWBLK, LATENT), lambda i: (0, i, 0)),
            pl.BlockSpec((NUM_TILES, ROWBLK), lambda i: (0, i)),
            pl.BlockSpec((ROWBLK, LATENT), lambda i: (i, 0)),
            pl.BlockSpec((ROWBLK, 1), lambda i: (i, 0)),
            pl.BlockSpec((ROWBLK, 1), lambda i: (i, 0)),
            pl.BlockSpec((1, 1), lambda i: (0, 0)),
            pl.BlockSpec((1, LATENT), lambda i: (0, 0)),
        ],
        out_specs=pl.BlockSpec((ROWBLK, LATENT), lambda i: (i, 0)),
        out_shape=jax.ShapeDtypeStruct((N, LATENT), jnp.float32),
    )(p, den, h, s, d, smax, bias.reshape(1, -1))


def _ew_body(s_hbm, d_hbm, sm_hbm, src_hbm, dst_hbm,
             rec_hbm, den_hbm,
             s_v, d_v, den_v, sm_v, src_all, dst_all, stg):
    c = lax.axis_index("c")
    sub = lax.axis_index("s")
    wid = c * 16 + sub
    base = wid * EPT

    pltpu.sync_copy(s_hbm, s_v)
    pltpu.sync_copy(d_hbm, d_v)
    pltpu.sync_copy(sm_hbm, sm_v)
    pltpu.sync_copy(src_hbm.at[pl.ds(base, EPT)], src_all)
    pltpu.sync_copy(dst_hbm.at[pl.ds(base, EPT)], dst_all)

    zero16 = jnp.zeros((16,), jnp.float32)

    def _zden(i, carry):
        den_v[pl.ds(i * 16, 16)] = zero16
        return carry
    lax.fori_loop(0, N // 16, _zden, 0)

    smax = sm_v[...]
    ngrp = CHUNK // 16

    def _chunk(j, carry):
        for t in range(ngrp):
            o = j * CHUNK + t * 16
            ro = j * REC + t * 16
            isrc = src_all[pl.ds(o, 16)]
            idst = dst_all[pl.ds(o, 16)]
            sv = plsc.load_gather(s_v, [isrc])
            dv = plsc.load_gather(d_v, [idst])
            t1 = smax + dv
            m = jnp.where(t1 > 0, t1, NEG * t1)
            a = sv + dv
            a = jnp.where(a > 0, a, NEG * a)
            ea = jnp.exp(a - m)
            plsc.addupdate_scatter(den_v, [idst], ea)
            stg[pl.ds(ro, 16)] = isrc
            stg[pl.ds(ro + CHUNK, 16)] = idst
            stg[pl.ds(ro + 2 * CHUNK, 16)] = plsc.bitcast(ea, jnp.int32)
        return carry

    lax.fori_loop(0, EPT // CHUNK, _chunk, 0)

    pltpu.sync_copy(stg, rec_hbm.at[pl.ds(wid * 3 * EPT, 3 * EPT)])
    pltpu.sync_copy(den_v, den_hbm.at[wid])


def _edge_weights(s, d, smax16, src, dst):
    mesh = plsc.VectorSubcoreMesh(core_axis_name="c", subcore_axis_name="s")
    f = pl.kernel(
        _ew_body,
        out_type=(
            jax.ShapeDtypeStruct((3 * E,), jnp.int32),
            jax.ShapeDtypeStruct((NUM_TILES, N), jnp.float32),
        ),
        mesh=mesh,
        scratch_types=(
            pltpu.VMEM((N,), jnp.float32),
            pltpu.VMEM((N,), jnp.float32),
            pltpu.VMEM((N,), jnp.float32),
            pltpu.VMEM((16,), jnp.float32),
            pltpu.VMEM((EPT,), jnp.int32),
            pltpu.VMEM((EPT,), jnp.int32),
            pltpu.VMEM((3 * EPT,), jnp.int32),
        ),
        compiler_params=pltpu.CompilerParams(needs_layout_passes=False),
    )
    return f(s, d, smax16, src, dst)


def _rows_body(h_hbm, rec_hbm, out_hbm, *refs):
    rows = refs[0:2]
    recs = refs[2:6]
    dstc = refs[6]
    out_sp = refs[7]
    gsem = refs[8:10]
    ssem = refs[10:12]
    rsem = refs[12:16]

    c = lax.axis_index("c")
    sub = lax.axis_index("s")
    wid = c * 16 + sub
    base_blk = wid * NC

    zero16 = jnp.zeros((16,), jnp.float32)

    def _zrows(e, carry):
        for f in range(LATENT // 16):
            rows[0][e, pl.ds(f * 16, 16)] = zero16
        return carry
    lax.fori_loop(0, CHUNK, _zrows, 0)

    myrow = sub * (N // 16)
    nz = (N // 16) // CHUNK
    for r in range(nz):
        pltpu.sync_copy(rows[0], out_sp.at[pl.ds(myrow + r * CHUNK, CHUNK)])
    rem = (N // 16) - nz * CHUNK
    if rem:
        pltpu.sync_copy(rows[0].at[pl.ds(0, rem)],
                        out_sp.at[pl.ds(myrow + nz * CHUNK, rem)])

    plsc.subcore_barrier()

    def _issue_rec(k, q):
        off = (base_blk + k) * REC
        pltpu.async_copy(rec_hbm.at[pl.ds(off, REC)], recs[q], rsem[q])

    def _wait_rec(q):
        pltpu.make_async_copy(rec_hbm.at[pl.ds(0, REC)], recs[q],
                              rsem[q]).wait()

    def _copy_dstc(q):
        for t in range(CHUNK // 16):
            dstc[q, pl.ds(t * 16, 16)] = recs[q][pl.ds(CHUNK + t * 16, 16)]

    def _issue_gather(q, r):
        pltpu.async_copy(h_hbm.at[recs[q].at[pl.ds(0, CHUNK)]], rows[r],
                         gsem[r])

    def _wait_gather(q, r):
        pltpu.make_async_copy(h_hbm.at[recs[q].at[pl.ds(0, CHUNK)]], rows[r],
                              gsem[r]).wait()

    def _issue_scatter(q, r):
        pltpu.async_copy(rows[r], out_sp.at[dstc.at[q]], ssem[r], add=True)

    def _drain_scatter(q, r):
        pltpu.make_async_copy(rows[r], out_sp.at[dstc.at[q]], ssem[r]).wait()

    def _scale(q, r):
        def _grp(i, carry):
            for v in range(4):
                e = i * 4 + v
                sca = plsc.bitcast(
                    plsc.load_gather(recs[q],
                                     [lax.broadcast(2 * CHUNK + e, (16,))]),
                    jnp.float32)
                for f in range(LATENT // 16):
                    rows[r][e, pl.ds(f * 16, 16)] = (
                        rows[r][e, pl.ds(f * 16, 16)] * sca)
            return carry
        lax.fori_loop(0, CHUNK // 4, _grp, 0)

    for i in range(3):
        _issue_rec(i, i)
    _wait_rec(0)
    _copy_dstc(0)
    _issue_gather(0, 0)

    def _quad(kk, carry):
        for u in range(4):
            k = kk * 4 + u
            q = u
            r = u % 2
            q1 = (u + 1) % 4
            r1 = (u + 1) % 2

            _wait_gather(q, r)

            @pl.when(k + 1 < NC)
            def _():
                _wait_rec(q1)

                @pl.when(k >= 1)
                def _():
                    _drain_scatter((u + 3) % 4, r1)

                _copy_dstc(q1)
                _issue_gather(q1, r1)

            _scale(q, r)
            _issue_scatter(q, r)

            @pl.when(k + 3 < NC)
            def _():
                _issue_rec(k + 3, (u + 3) % 4)
        return carry

    lax.fori_loop(0, NC // 4, _quad, 0)

    _drain_scatter(2, 0)
    _drain_scatter(3, 1)

    plsc.subcore_barrier()

    pltpu.sync_copy(out_sp.at[pl.ds(myrow, N // 16)],
                    out_hbm.at[c, pl.ds(myrow, N // 16)])


def _edge_rows(h, rec):
    mesh = plsc.VectorSubcoreMesh(core_axis_name="c", subcore_axis_name="s")
    scratch = (
        [pltpu.VMEM((CHUNK, LATENT), jnp.float32) for _ in range(2)]
        + [pltpu.VMEM((REC,), jnp.int32) for _ in range(4)]
        + [pltpu.VMEM((4, CHUNK), jnp.int32)]
        + [pltpu.VMEM_SHARED((N, LATENT), jnp.float32)]
        + [pltpu.SemaphoreType.DMA for _ in range(8)]
    )
    f = pl.kernel(
        _rows_body,
        out_type=jax.ShapeDtypeStruct((2, N, LATENT), jnp.float32),
        mesh=mesh,
        scratch_types=tuple(scratch),
        compiler_params=pltpu.CompilerParams(needs_layout_passes=False),
    )
    return f(h, rec)


def kernel(z, edge_index, params):
    src = edge_index[0]
    dst = edge_index[1]
    x = _fc(z, params["fc_W"], params["fc_b"]).reshape(N, LATENT)
    convs = params["convs"]
    for i, p in enumerate(convs):
        h, s, d, smax = _pre(x, p["W"], p["a_src"], p["a_dst"])
        smax16 = jnp.broadcast_to(smax.reshape(1), (16,))
        rec, den = _edge_weights(s.reshape(N), d.reshape(N), smax16, src, dst)
        part = _edge_rows(h, rec)
        x = _combine(part, den, h, s, d, smax, p["b"],
                     relu=(i < len(convs) - 1))
    return x


# speedup vs baseline: 2.8840x; 1.0239x over previous
import functools

import jax
import jax.numpy as jnp
from jax import lax
from jax.experimental import pallas as pl
from jax.experimental.pallas import tpu as pltpu
from jax.experimental.pallas import tpu_sc as plsc

B = 64
NUM_NODES = 196
N = B * NUM_NODES
E = 401408
LATENT = 128
HALF = LATENT // 2
NEG = 0.2

NUM_TILES = 32
EPT = E // NUM_TILES
CHUNK = 64
REC = 3 * CHUNK
NC = EPT // CHUNK
ROWBLK = 896


def _leaky(x):
    return jnp.where(x > 0, x, NEG * x)


def _fc_body(z_ref, w_ref, b_ref, o_ref):
    acc = lax.dot_general(z_ref[...], w_ref[...],
                          (((1,), (1,)), ((), ())),
                          preferred_element_type=jnp.float32)
    o_ref[...] = jnp.maximum(acc + b_ref[...], 0.0)


def _fc(z, fc_w, fc_b):
    nblk = 49
    blk = (NUM_NODES * LATENT) // nblk
    return pl.pallas_call(
        _fc_body,
        grid=(nblk,),
        in_specs=[
            pl.BlockSpec((B, LATENT), lambda i: (0, 0)),
            pl.BlockSpec((blk, LATENT), lambda i: (i, 0)),
            pl.BlockSpec((1, blk), lambda i: (0, i)),
        ],
        out_specs=pl.BlockSpec((B, blk), lambda i: (0, i)),
        out_shape=jax.ShapeDtypeStruct((B, NUM_NODES * LATENT), jnp.float32),
    )(z, fc_w, fc_b.reshape(1, -1))


def _pre_body(x_ref, w_ref, asrc_ref, adst_ref,
              h_ref, s_ref, d_ref, sm_ref):
    i = pl.program_id(0)
    h = lax.dot_general(x_ref[...], w_ref[...],
                        (((1,), (0,)), ((), ())),
                        preferred_element_type=jnp.float32)
    h_ref[...] = h
    s = jnp.sum(h * asrc_ref[...], axis=1, keepdims=True)
    d = jnp.sum(h * adst_ref[...], axis=1, keepdims=True)
    s_ref[...] = s
    d_ref[...] = d

    @pl.when(i == 0)
    def _():
        sm_ref[...] = jnp.full((1, 1), -jnp.inf, jnp.float32)

    sm_ref[...] = jnp.maximum(sm_ref[...], jnp.max(s))


def _pre(x, w, a_src, a_dst):
    nblk = N // ROWBLK
    return pl.pallas_call(
        _pre_body,
        grid=(nblk,),
        in_specs=[
            pl.BlockSpec((ROWBLK, LATENT), lambda i: (i, 0)),
            pl.BlockSpec((LATENT, LATENT), lambda i: (0, 0)),
            pl.BlockSpec((1, LATENT), lambda i: (0, 0)),
            pl.BlockSpec((1, LATENT), lambda i: (0, 0)),
        ],
        out_specs=[
            pl.BlockSpec((ROWBLK, LATENT), lambda i: (i, 0)),
            pl.BlockSpec((ROWBLK, 1), lambda i: (i, 0)),
            pl.BlockSpec((ROWBLK, 1), lambda i: (i, 0)),
            pl.BlockSpec((1, 1), lambda i: (0, 0)),
        ],
        out_shape=[
            jax.ShapeDtypeStruct((N, LATENT), jnp.float32),
            jax.ShapeDtypeStruct((N, 1), jnp.float32),
            jax.ShapeDtypeStruct((N, 1), jnp.float32),
            jax.ShapeDtypeStruct((1, 1), jnp.float32),
        ],
    )(x, w, a_src.reshape(1, -1), a_dst.reshape(1, -1))


def _combine_body(relu, p_ref, den_ref, h_ref, s_ref, d_ref,
                  sm_ref, b_ref, o_ref):
    s = s_ref[...]
    d = d_ref[...]
    smax = sm_ref[0, 0]
    m = _leaky(smax + d)
    ea_self = jnp.exp(_leaky(s + d) - m)
    den_tot = jnp.sum(den_ref[...], axis=0)[:, None] + ea_self
    num = p_ref[0] + p_ref[1] + ea_self * h_ref[...]
    out = num / den_tot + b_ref[...]
    if relu:
        out = jnp.maximum(out, 0.0)
    o_ref[...] = out


def _combine(p, den, h, s, d, smax, bias, relu):
    nblk = N // ROWBLK
    return pl.pallas_call(
        functools.partial(_combine_body, relu),
        grid=(nblk,),
        in_specs=[
            pl.BlockSpec((2, ROWBLK, LATENT), lambda i: (0, i, 0)),
            pl.BlockSpec((NUM_TILES, ROWBLK), lambda i: (0, i)),
            pl.BlockSpec((ROWBLK, LATENT), lambda i: (i, 0)),
            pl.BlockSpec((ROWBLK, 1), lambda i: (i, 0)),
            pl.BlockSpec((ROWBLK, 1), lambda i: (i, 0)),
            pl.BlockSpec((1, 1), lambda i: (0, 0)),
            pl.BlockSpec((1, LATENT), lambda i: (0, 0)),
        ],
        out_specs=pl.BlockSpec((ROWBLK, LATENT), lambda i: (i, 0)),
        out_shape=jax.ShapeDtypeStruct((N, LATENT), jnp.float32),
    )(p, den, h, s, d, smax, bias.reshape(1, -1))


def _mid_body(p_ref, den_ref, h_ref, s_ref, d_ref, sm_ref, b_ref,
              w2_ref, as2_ref, ad2_ref,
              h2_ref, s2_ref, d2_ref, sm2_ref):
    i = pl.program_id(0)
    s = s_ref[...]
    d = d_ref[...]
    smax = sm_ref[0, 0]
    m = _leaky(smax + d)
    ea_self = jnp.exp(_leaky(s + d) - m)
    den_tot = jnp.sum(den_ref[...], axis=0)[:, None] + ea_self
    num = p_ref[0] + p_ref[1] + ea_self * h_ref[...]
    x = jnp.maximum(num / den_tot + b_ref[...], 0.0)
    h2 = lax.dot_general(x, w2_ref[...],
                         (((1,), (0,)), ((), ())),
                         preferred_element_type=jnp.float32)
    h2_ref[...] = h2
    s2 = jnp.sum(h2 * as2_ref[...], axis=1, keepdims=True)
    d2 = jnp.sum(h2 * ad2_ref[...], axis=1, keepdims=True)
    s2_ref[...] = s2
    d2_ref[...] = d2

    @pl.when(i == 0)
    def _():
        sm2_ref[...] = jnp.full((1, 1), -jnp.inf, jnp.float32)

    sm2_ref[...] = jnp.maximum(sm2_ref[...], jnp.max(s2))


def _mid(p, den, h, s, d, smax, bias, w2, a_src2, a_dst2):
    nblk = N // ROWBLK
    return pl.pallas_call(
        _mid_body,
        grid=(nblk,),
        in_specs=[
            pl.BlockSpec((2, ROWBLK, LATENT), lambda i: (0, i, 0)),
            pl.BlockSpec((NUM_TILES, ROWBLK), lambda i: (0, i)),
            pl.BlockSpec((ROWBLK, LATENT), lambda i: (i, 0)),
            pl.BlockSpec((ROWBLK, 1), lambda i: (i, 0)),
            pl.BlockSpec((ROWBLK, 1), lambda i: (i, 0)),
            pl.BlockSpec((1, 1), lambda i: (0, 0)),
            pl.BlockSpec((1, LATENT), lambda i: (0, 0)),
            pl.BlockSpec((LATENT, LATENT), lambda i: (0, 0)),
            pl.BlockSpec((1, LATENT), lambda i: (0, 0)),
            pl.BlockSpec((1, LATENT), lambda i: (0, 0)),
        ],
        out_specs=[
            pl.BlockSpec((ROWBLK, LATENT), lambda i: (i, 0)),
            pl.BlockSpec((ROWBLK, 1), lambda i: (i, 0)),
            pl.BlockSpec((ROWBLK, 1), lambda i: (i, 0)),
            pl.BlockSpec((1, 1), lambda i: (0, 0)),
        ],
        out_shape=[
            jax.ShapeDtypeStruct((N, LATENT), jnp.float32),
            jax.ShapeDtypeStruct((N, 1), jnp.float32),
            jax.ShapeDtypeStruct((N, 1), jnp.float32),
            jax.ShapeDtypeStruct((1, 1), jnp.float32),
        ],
    )(p, den, h, s, d, smax, bias.reshape(1, -1), w2,
      a_src2.reshape(1, -1), a_dst2.reshape(1, -1))


def _ew_body(s_hbm, d_hbm, sm_hbm, src_hbm, dst_hbm,
             rec_hbm, den_hbm,
             s_v, d_v, den_v, sm_v, src_all, dst_all, stg):
    c = lax.axis_index("c")
    sub = lax.axis_index("s")
    wid = c * 16 + sub
    base = wid * EPT

    pltpu.sync_copy(s_hbm, s_v)
    pltpu.sync_copy(d_hbm, d_v)
    pltpu.sync_copy(sm_hbm, sm_v)
    pltpu.sync_copy(src_hbm.at[pl.ds(base, EPT)], src_all)
    pltpu.sync_copy(dst_hbm.at[pl.ds(base, EPT)], dst_all)

    zero16 = jnp.zeros((16,), jnp.float32)

    def _zden(i, carry):
        den_v[pl.ds(i * 16, 16)] = zero16
        return carry
    lax.fori_loop(0, N // 16, _zden, 0)

    smax = sm_v[...]
    ngrp = CHUNK // 16

    def _chunk(j, carry):
        for t in range(ngrp):
            o = j * CHUNK + t * 16
            ro = j * REC + t * 16
            isrc = src_all[pl.ds(o, 16)]
            idst = dst_all[pl.ds(o, 16)]
            sv = plsc.load_gather(s_v, [isrc])
            dv = plsc.load_gather(d_v, [idst])
            t1 = smax + dv
            m = jnp.where(t1 > 0, t1, NEG * t1)
            a = sv + dv
            a = jnp.where(a > 0, a, NEG * a)
            ea = jnp.exp(a - m)
            plsc.addupdate_scatter(den_v, [idst], ea)
            stg[pl.ds(ro, 16)] = isrc
            stg[pl.ds(ro + CHUNK, 16)] = idst
            stg[pl.ds(ro + 2 * CHUNK, 16)] = plsc.bitcast(ea, jnp.int32)
        return carry

    lax.fori_loop(0, EPT // CHUNK, _chunk, 0)

    pltpu.sync_copy(stg, rec_hbm.at[pl.ds(wid * 3 * EPT, 3 * EPT)])
    pltpu.sync_copy(den_v, den_hbm.at[wid])


def _edge_weights(s, d, smax16, src, dst):
    mesh = plsc.VectorSubcoreMesh(core_axis_name="c", subcore_axis_name="s")
    f = pl.kernel(
        _ew_body,
        out_type=(
            jax.ShapeDtypeStruct((3 * E,), jnp.int32),
            jax.ShapeDtypeStruct((NUM_TILES, N), jnp.float32),
        ),
        mesh=mesh,
        scratch_types=(
            pltpu.VMEM((N,), jnp.float32),
            pltpu.VMEM((N,), jnp.float32),
            pltpu.VMEM((N,), jnp.float32),
            pltpu.VMEM((16,), jnp.float32),
            pltpu.VMEM((EPT,), jnp.int32),
            pltpu.VMEM((EPT,), jnp.int32),
            pltpu.VMEM((3 * EPT,), jnp.int32),
        ),
        compiler_params=pltpu.CompilerParams(needs_layout_passes=False),
    )
    return f(s, d, smax16, src, dst)


def _rows_body(h_hbm, rec_hbm, out_hbm, *refs):
    rows = refs[0:2]
    recs = refs[2:6]
    dstc = refs[6]
    out_sp = refs[7]
    gsem = refs[8:10]
    ssem = refs[10:12]
    rsem = refs[12:16]

    c = lax.axis_index("c")
    sub = lax.axis_index("s")
    wid = c * 16 + sub
    base_blk = wid * NC

    zero16 = jnp.zeros((16,), jnp.float32)

    def _zrows(e, carry):
        for f in range(LATENT // 16):
            rows[0][e, pl.ds(f * 16, 16)] = zero16
        return carry
    lax.fori_loop(0, CHUNK, _zrows, 0)

    myrow = sub * (N // 16)
    nz = (N // 16) // CHUNK
    for r in range(nz):
        pltpu.sync_copy(rows[0], out_sp.at[pl.ds(myrow + r * CHUNK, CHUNK)])
    rem = (N // 16) - nz * CHUNK
    if rem:
        pltpu.sync_copy(rows[0].at[pl.ds(0, rem)],
                        out_sp.at[pl.ds(myrow + nz * CHUNK, rem)])

    plsc.subcore_barrier()

    def _issue_rec(k, q):
        off = (base_blk + k) * REC
        pltpu.async_copy(rec_hbm.at[pl.ds(off, REC)], recs[q], rsem[q])

    def _wait_rec(q):
        pltpu.make_async_copy(rec_hbm.at[pl.ds(0, REC)], recs[q],
                              rsem[q]).wait()

    def _copy_dstc(q):
        for t in range(CHUNK // 16):
            dstc[q, pl.ds(t * 16, 16)] = recs[q][pl.ds(CHUNK + t * 16, 16)]

    def _issue_gather(q, r):
        pltpu.async_copy(h_hbm.at[recs[q].at[pl.ds(0, CHUNK)]], rows[r],
                         gsem[r])

    def _wait_gather(q, r):
        pltpu.make_async_copy(h_hbm.at[recs[q].at[pl.ds(0, CHUNK)]], rows[r],
                              gsem[r]).wait()

    def _issue_scatter(q, r):
        pltpu.async_copy(rows[r], out_sp.at[dstc.at[q]], ssem[r], add=True)

    def _drain_scatter(q, r):
        pltpu.make_async_copy(rows[r], out_sp.at[dstc.at[q]], ssem[r]).wait()

    def _scale(q, r):
        def _grp(i, carry):
            for v in range(4):
                e = i * 4 + v
                sca = plsc.bitcast(
                    plsc.load_gather(recs[q],
                                     [lax.broadcast(2 * CHUNK + e, (16,))]),
                    jnp.float32)
                for f in range(LATENT // 16):
                    rows[r][e, pl.ds(f * 16, 16)] = (
                        rows[r][e, pl.ds(f * 16, 16)] * sca)
            return carry
        lax.fori_loop(0, CHUNK // 4, _grp, 0)

    for i in range(3):
        _issue_rec(i, i)
    _wait_rec(0)
    _copy_dstc(0)
    _issue_gather(0, 0)

    def _quad(kk, carry):
        for u in range(4):
            k = kk * 4 + u
            q = u
            r = u % 2
            q1 = (u + 1) % 4
            r1 = (u + 1) % 2

            _wait_gather(q, r)

            @pl.when(k + 1 < NC)
            def _():
                _wait_rec(q1)

                @pl.when(k >= 1)
                def _():
                    _drain_scatter((u + 3) % 4, r1)

                _copy_dstc(q1)
                _issue_gather(q1, r1)

            _scale(q, r)
            _issue_scatter(q, r)

            @pl.when(k + 3 < NC)
            def _():
                _issue_rec(k + 3, (u + 3) % 4)
        return carry

    lax.fori_loop(0, NC // 4, _quad, 0)

    _drain_scatter(2, 0)
    _drain_scatter(3, 1)

    plsc.subcore_barrier()

    pltpu.sync_copy(out_sp.at[pl.ds(myrow, N // 16)],
                    out_hbm.at[c, pl.ds(myrow, N // 16)])


def _edge_rows(h, rec):
    mesh = plsc.VectorSubcoreMesh(core_axis_name="c", subcore_axis_name="s")
    scratch = (
        [pltpu.VMEM((CHUNK, LATENT), jnp.float32) for _ in range(2)]
        + [pltpu.VMEM((REC,), jnp.int32) for _ in range(4)]
        + [pltpu.VMEM((4, CHUNK), jnp.int32)]
        + [pltpu.VMEM_SHARED((N, LATENT), jnp.float32)]
        + [pltpu.SemaphoreType.DMA for _ in range(8)]
    )
    f = pl.kernel(
        _rows_body,
        out_type=jax.ShapeDtypeStruct((2, N, LATENT), jnp.float32),
        mesh=mesh,
        scratch_types=tuple(scratch),
        compiler_params=pltpu.CompilerParams(needs_layout_passes=False),
    )
    return f(h, rec)


def kernel(z, edge_index, params):
    src = edge_index[0]
    dst = edge_index[1]
    x = _fc(z, params["fc_W"], params["fc_b"]).reshape(N, LATENT)
    convs = params["convs"]
    h, s, d, smax = _pre(x, convs[0]["W"], convs[0]["a_src"],
                         convs[0]["a_dst"])
    for i, p in enumerate(convs):
        smax16 = jnp.broadcast_to(smax.reshape(1), (16,))
        rec, den = _edge_weights(s.reshape(N), d.reshape(N), smax16, src, dst)
        part = _edge_rows(h, rec)
        if i < len(convs) - 1:
            p2 = convs[i + 1]
            h, s, d, smax = _mid(part, den, h, s, d, smax, p["b"],
                                 p2["W"], p2["a_src"], p2["a_dst"])
        else:
            x = _combine(part, den, h, s, d, smax, p["b"], relu=False)
    return x
